# baseline (speedup 1.0000x reference)
import functools

import jax
import jax.numpy as jnp
from jax import lax
from jax.experimental import pallas as pl
from jax.experimental.pallas import tpu as pltpu
from jax.experimental.pallas import tpu_sc as plsc

_T = 2048
_D = 2048
_F = 2048
_E = 8
_SOFTCAP = 30.0

_BT = 128
_S_PAD = 2 * _T + _E * _BT
_NB = _S_PAD // _BT
_NB_OUT = 64
_BF = 1024

_NC = 2
_NS = 16
_NW = _NC * _NS
_PER_W = _T // _NW
_CH = 32


def _excl_cumsum_tokens(h):
    c = h
    s = 1
    while s < _T:
        z = jnp.zeros((s, 128), jnp.float32)
        c = c + jnp.concatenate([z, c[:-s]], axis=0)
        s *= 2
    return c - h


def _router_kernel(hid_ref, gw_ref, p1_ref, p2_ref, w1_ref, w2_ref,
                   be_ref, bv_ref):
    x = hid_ref[...]
    gw = gw_ref[...]
    logits = jnp.dot(x, gw, preferred_element_type=jnp.float32)
    lane = lax.broadcasted_iota(jnp.int32, (_T, 128), 1)
    neg = jnp.float32(-1e30)
    logits = jnp.where(lane < _E, _SOFTCAP * jnp.tanh(logits / _SOFTCAP), neg)
    v1 = jnp.max(logits, axis=-1, keepdims=True)
    i1 = jnp.min(jnp.where(logits == v1, lane, 127), axis=-1, keepdims=True)
    l2 = jnp.where(lane == i1, neg, logits)
    v2 = jnp.max(l2, axis=-1, keepdims=True)
    i2 = jnp.min(jnp.where(l2 == v2, lane, 127), axis=-1, keepdims=True)
    e2v = jnp.exp(v2 - v1)
    w1_ref[...] = 1.0 / (1.0 + e2v)
    w2_ref[...] = e2v / (1.0 + e2v)

    h1 = (lane == i1).astype(jnp.float32)
    h2 = (lane == i2).astype(jnp.float32)
    cnt1 = jnp.sum(h1, axis=0, keepdims=True)
    cnt2 = jnp.sum(h2, axis=0, keepdims=True)
    cnt = cnt1 + cnt2
    gpad = jnp.ceil(cnt / _BT) * _BT
    r_i = lax.broadcasted_iota(jnp.int32, (128, 128), 0)
    c_i = lax.broadcasted_iota(jnp.int32, (128, 128), 1)
    lt = (r_i < c_i).astype(jnp.float32)
    pad_off = jnp.dot(gpad, lt, preferred_element_type=jnp.float32)

    c1x = _excl_cumsum_tokens(h1)
    c2x = _excl_cumsum_tokens(h2)
    rank1 = jnp.sum(c1x * h1, axis=-1, keepdims=True)
    rank2 = jnp.sum(c2x * h2, axis=-1, keepdims=True)
    off1 = jnp.sum(pad_off * h1, axis=-1, keepdims=True)
    off2 = jnp.sum(pad_off * h2, axis=-1, keepdims=True)
    c1_at_e2 = jnp.sum(cnt1 * h2, axis=-1, keepdims=True)
    p1_ref[...] = (off1 + rank1).astype(jnp.int32)
    p2_ref[...] = (off2 + c1_at_e2 + rank2).astype(jnp.int32)

    b_start = (lax.broadcasted_iota(jnp.int32, (_NB_OUT, 128), 0)
               .astype(jnp.float32) * _BT)
    lane_e = lax.broadcasted_iota(jnp.int32, (_NB_OUT, 128), 1)
    sel = (b_start >= pad_off) & (b_start < pad_off + gpad) & (lane_e < _E)
    be = 7 - jnp.sum(jnp.where(sel, (7 - lane_e).astype(jnp.float32), 0.0),
                     axis=-1, keepdims=True)
    bv = jnp.sum(jnp.where(sel & (b_start < pad_off + cnt), 1.0, 0.0),
                 axis=-1, keepdims=True)
    be_ref[...] = be.astype(jnp.int32)
    bv_ref[...] = bv.astype(jnp.int32)


def _router(hidden, gate_w_pad):
    out_shapes = (
        jax.ShapeDtypeStruct((_T, 1), jnp.int32),
        jax.ShapeDtypeStruct((_T, 1), jnp.int32),
        jax.ShapeDtypeStruct((_T, 1), jnp.float32),
        jax.ShapeDtypeStruct((_T, 1), jnp.float32),
        jax.ShapeDtypeStruct((_NB_OUT, 1), jnp.int32),
        jax.ShapeDtypeStruct((_NB_OUT, 1), jnp.int32),
    )
    return pl.pallas_call(
        _router_kernel,
        out_shape=out_shapes,
    )(hidden, gate_w_pad)


def _dispatch_kernel(x_hbm, p1_hbm, p2_hbm, out_hbm,
                     idx1_v, idx2_v, rows_v, sem1, sem2):
    wid = lax.axis_index("s") * _NC + lax.axis_index("c")
    base = wid * _PER_W
    for c in range(_PER_W // _CH):
        b = base + c * _CH
        pltpu.sync_copy(p1_hbm.at[pl.ds(b, _CH)], idx1_v)
        pltpu.sync_copy(p2_hbm.at[pl.ds(b, _CH)], idx2_v)
        pltpu.sync_copy(x_hbm.at[pl.ds(b, _CH)], rows_v)
        cp1 = pltpu.make_async_copy(rows_v, out_hbm.at[idx1_v], sem1)
        cp2 = pltpu.make_async_copy(rows_v, out_hbm.at[idx2_v], sem2)
        cp1.start()
        cp2.start()
        cp1.wait()
        cp2.wait()


def _dispatch(hidden16, p1, p2):
    mesh = plsc.VectorSubcoreMesh(core_axis_name="c", subcore_axis_name="s")
    f = functools.partial(
        pl.kernel,
        out_type=jax.ShapeDtypeStruct((_S_PAD, _D), jnp.float32),
        mesh=mesh,
        scratch_types=[
            pltpu.VMEM((_CH,), jnp.int32),
            pltpu.VMEM((_CH,), jnp.int32),
            pltpu.VMEM((_CH, _D), jnp.float32),
            pltpu.SemaphoreType.DMA,
            pltpu.SemaphoreType.DMA,
        ],
    )(_dispatch_kernel)
    return f(hidden16, p1, p2)


_DK = _D // 2


def _gemm1_kernel(be_ref, bv_ref, x_ref, wga_ref, wgb_ref, wua_ref, wub_ref,
                  h_ref, wg16_ref, wu16_ref):
    b = pl.program_id(1)
    new_w = jnp.logical_or(b == 0, be_ref[jnp.maximum(b - 1, 0)] != be_ref[b])

    @pl.when(new_w)
    def _():
        wg16_ref[:_DK] = wga_ref[0].astype(jnp.bfloat16)
        wg16_ref[_DK:] = wgb_ref[0].astype(jnp.bfloat16)
        wu16_ref[:_DK] = wua_ref[0].astype(jnp.bfloat16)
        wu16_ref[_DK:] = wub_ref[0].astype(jnp.bfloat16)

    @pl.when(bv_ref[b] == 1)
    def _():
        x = x_ref[...].astype(jnp.bfloat16)
        g = jnp.dot(x, wg16_ref[...], preferred_element_type=jnp.float32)
        u = jnp.dot(x, wu16_ref[...], preferred_element_type=jnp.float32)
        h_ref[...] = (jax.nn.gelu(g) * u).astype(jnp.bfloat16)


def _gemm1(be, bv, x_sorted, w_gate, w_up):
    nf = _F // _BF
    wspec_a = pl.BlockSpec((1, _DK, _BF), lambda f, b, be, bv: (be[b], 0, f))
    wspec_b = pl.BlockSpec((1, _DK, _BF), lambda f, b, be, bv: (be[b], 1, f))
    grid_spec = pltpu.PrefetchScalarGridSpec(
        num_scalar_prefetch=2,
        grid=(nf, _NB),
        in_specs=[
            pl.BlockSpec((_BT, _D), lambda f, b, be, bv: (b, 0)),
            wspec_a, wspec_b, wspec_a, wspec_b,
        ],
        out_specs=pl.BlockSpec((_BT, _BF), lambda f, b, be, bv: (b, f)),
        scratch_shapes=[
            pltpu.VMEM((_D, _BF), jnp.bfloat16),
            pltpu.VMEM((_D, _BF), jnp.bfloat16),
        ],
    )
    return pl.pallas_call(
        _gemm1_kernel,
        grid_spec=grid_spec,
        out_shape=jax.ShapeDtypeStruct((_S_PAD, _F), jnp.bfloat16),
    )(be, bv, x_sorted, w_gate, w_gate, w_up, w_up)


def _gemm2_kernel(be_ref, bv_ref, h_ref, wda_ref, wdb_ref, y_ref, wd16_ref):
    b = pl.program_id(0)
    new_w = jnp.logical_or(b == 0, be_ref[jnp.maximum(b - 1, 0)] != be_ref[b])

    @pl.when(new_w)
    def _():
        wd16_ref[:_DK] = wda_ref[0].astype(jnp.bfloat16)
        wd16_ref[_DK:] = wdb_ref[0].astype(jnp.bfloat16)

    @pl.when(bv_ref[b] == 1)
    def _():
        y_ref[...] = jnp.dot(h_ref[...], wd16_ref[...],
                             preferred_element_type=jnp.float32)


def _gemm2(be, bv, h, w_down):
    wspec_a = pl.BlockSpec((1, _DK, _D), lambda b, be, bv: (be[b], 0, 0))
    wspec_b = pl.BlockSpec((1, _DK, _D), lambda b, be, bv: (be[b], 1, 0))
    grid_spec = pltpu.PrefetchScalarGridSpec(
        num_scalar_prefetch=2,
        grid=(_NB,),
        in_specs=[
            pl.BlockSpec((_BT, _F), lambda b, be, bv: (b, 0)),
            wspec_a, wspec_b,
        ],
        out_specs=pl.BlockSpec((_BT, _D), lambda b, be, bv: (b, 0)),
        scratch_shapes=[
            pltpu.VMEM((_F, _D), jnp.bfloat16),
        ],
    )
    return pl.pallas_call(
        _gemm2_kernel,
        grid_spec=grid_spec,
        out_shape=jax.ShapeDtypeStruct((_S_PAD, _D), jnp.float32),
    )(be, bv, h, w_down, w_down)


def _gather_kernel(y_hbm, p1_hbm, p2_hbm, y1_hbm, y2_hbm,
                   idx_v, rows_v, sem):
    wid = lax.axis_index("s") * _NC + lax.axis_index("c")
    base = wid * _PER_W
    for c in range(_PER_W // _CH):
        b = base + c * _CH
        pltpu.sync_copy(p1_hbm.at[pl.ds(b, _CH)], idx_v)
        pltpu.make_async_copy(y_hbm.at[idx_v], rows_v, sem).start()
        pltpu.make_async_copy(y_hbm.at[idx_v], rows_v, sem).wait()
        pltpu.sync_copy(rows_v, y1_hbm.at[pl.ds(b, _CH)])
        pltpu.sync_copy(p2_hbm.at[pl.ds(b, _CH)], idx_v)
        pltpu.make_async_copy(y_hbm.at[idx_v], rows_v, sem).start()
        pltpu.make_async_copy(y_hbm.at[idx_v], rows_v, sem).wait()
        pltpu.sync_copy(rows_v, y2_hbm.at[pl.ds(b, _CH)])


def _gather(y, p1, p2):
    mesh = plsc.VectorSubcoreMesh(core_axis_name="c", subcore_axis_name="s")
    f = functools.partial(
        pl.kernel,
        out_type=(
            jax.ShapeDtypeStruct((_T, _D), jnp.float32),
            jax.ShapeDtypeStruct((_T, _D), jnp.float32),
        ),
        mesh=mesh,
        scratch_types=[
            pltpu.VMEM((_CH,), jnp.int32),
            pltpu.VMEM((_CH, _D), jnp.float32),
            pltpu.SemaphoreType.DMA,
        ],
    )(_gather_kernel)
    return f(y, p1, p2)


def _combine_kernel(w1_ref, w2_ref, y1_ref, y2_ref, out_ref):
    out_ref[...] = w1_ref[...] * y1_ref[...] + w2_ref[...] * y2_ref[...]


def _combine(w1, w2, y1, y2):
    bt = 256
    grid_spec = pl.GridSpec(
        grid=(_T // bt,),
        in_specs=[
            pl.BlockSpec((bt, 1), lambda i: (i, 0)),
            pl.BlockSpec((bt, 1), lambda i: (i, 0)),
            pl.BlockSpec((bt, _D), lambda i: (i, 0)),
            pl.BlockSpec((bt, _D), lambda i: (i, 0)),
        ],
        out_specs=pl.BlockSpec((bt, _D), lambda i: (i, 0)),
    )
    return pl.pallas_call(
        _combine_kernel,
        grid_spec=grid_spec,
        out_shape=jax.ShapeDtypeStruct((_T, _D), jnp.float32),
    )(w1, w2, y1, y2)


def kernel(hidden_states, gate_w, w_gate, w_up, w_down):
    gw_pad = jnp.pad(gate_w, ((0, 0), (0, 128 - _E)))
    p1, p2, w1, w2, be, bv = _router(hidden_states, gw_pad)
    p1f = p1.reshape(_T)
    p2f = p2.reshape(_T)
    be = be.reshape(_NB_OUT)[:_NB]
    bv = bv.reshape(_NB_OUT)[:_NB]
    x_sorted = _dispatch(hidden_states, p1f, p2f)
    h = _gemm1(be, bv, x_sorted, w_gate, w_up)
    y = _gemm2(be, bv, h, w_down)
    y1, y2 = _gather(y, p1f, p2f)
    return _combine(w1, w2, y1, y2)


# speedup vs baseline: 1.0387x; 1.0387x over previous
import functools

import jax
import jax.numpy as jnp
from jax import lax
from jax.experimental import pallas as pl
from jax.experimental.pallas import tpu as pltpu
from jax.experimental.pallas import tpu_sc as plsc

_T = 2048
_D = 2048
_F = 2048
_E = 8
_SOFTCAP = 30.0

_BT = 128
_S_PAD = 2 * _T + _E * _BT
_NB = _S_PAD // _BT
_NB_OUT = 64
_BF = 1024

_NC = 2
_NS = 16
_NW = _NC * _NS
_PER_W = _T // _NW
_CH = 32


def _excl_cumsum_tokens(h):
    c = h
    s = 1
    while s < _T:
        z = jnp.zeros((s, 128), jnp.float32)
        c = c + jnp.concatenate([z, c[:-s]], axis=0)
        s *= 2
    return c - h


def _router_kernel(hid_ref, gw_ref, p1_ref, p2_ref, w1_ref, w2_ref,
                   be_ref, bv_ref):
    x = hid_ref[...]
    gw = gw_ref[...]
    logits = jnp.dot(x, gw, preferred_element_type=jnp.float32)
    lane = lax.broadcasted_iota(jnp.int32, (_T, 128), 1)
    neg = jnp.float32(-1e30)
    logits = jnp.where(lane < _E, _SOFTCAP * jnp.tanh(logits / _SOFTCAP), neg)
    v1 = jnp.max(logits, axis=-1, keepdims=True)
    i1 = jnp.min(jnp.where(logits == v1, lane, 127), axis=-1, keepdims=True)
    l2 = jnp.where(lane == i1, neg, logits)
    v2 = jnp.max(l2, axis=-1, keepdims=True)
    i2 = jnp.min(jnp.where(l2 == v2, lane, 127), axis=-1, keepdims=True)
    e2v = jnp.exp(v2 - v1)
    w1_ref[...] = 1.0 / (1.0 + e2v)
    w2_ref[...] = e2v / (1.0 + e2v)

    h1 = (lane == i1).astype(jnp.float32)
    h2 = (lane == i2).astype(jnp.float32)
    cnt1 = jnp.sum(h1, axis=0, keepdims=True)
    cnt2 = jnp.sum(h2, axis=0, keepdims=True)
    cnt = cnt1 + cnt2
    gpad = jnp.ceil(cnt / _BT) * _BT
    r_i = lax.broadcasted_iota(jnp.int32, (128, 128), 0)
    c_i = lax.broadcasted_iota(jnp.int32, (128, 128), 1)
    lt = (r_i < c_i).astype(jnp.float32)
    pad_off = jnp.dot(gpad, lt, preferred_element_type=jnp.float32)

    c1x = _excl_cumsum_tokens(h1)
    c2x = _excl_cumsum_tokens(h2)
    rank1 = jnp.sum(c1x * h1, axis=-1, keepdims=True)
    rank2 = jnp.sum(c2x * h2, axis=-1, keepdims=True)
    off1 = jnp.sum(pad_off * h1, axis=-1, keepdims=True)
    off2 = jnp.sum(pad_off * h2, axis=-1, keepdims=True)
    c1_at_e2 = jnp.sum(cnt1 * h2, axis=-1, keepdims=True)
    p1_ref[...] = (off1 + rank1).astype(jnp.int32)
    p2_ref[...] = (off2 + c1_at_e2 + rank2).astype(jnp.int32)

    b_start = (lax.broadcasted_iota(jnp.int32, (_NB_OUT, 128), 0)
               .astype(jnp.float32) * _BT)
    lane_e = lax.broadcasted_iota(jnp.int32, (_NB_OUT, 128), 1)
    sel = (b_start >= pad_off) & (b_start < pad_off + gpad) & (lane_e < _E)
    be = 7 - jnp.sum(jnp.where(sel, (7 - lane_e).astype(jnp.float32), 0.0),
                     axis=-1, keepdims=True)
    bv = jnp.sum(jnp.where(sel & (b_start < pad_off + cnt), 1.0, 0.0),
                 axis=-1, keepdims=True)
    be_ref[...] = be.astype(jnp.int32)
    bv_ref[...] = bv.astype(jnp.int32)


def _router(hidden, gate_w_pad):
    out_shapes = (
        jax.ShapeDtypeStruct((_T, 1), jnp.int32),
        jax.ShapeDtypeStruct((_T, 1), jnp.int32),
        jax.ShapeDtypeStruct((_T, 1), jnp.float32),
        jax.ShapeDtypeStruct((_T, 1), jnp.float32),
        jax.ShapeDtypeStruct((_NB_OUT, 1), jnp.int32),
        jax.ShapeDtypeStruct((_NB_OUT, 1), jnp.int32),
    )
    return pl.pallas_call(
        _router_kernel,
        out_shape=out_shapes,
    )(hidden, gate_w_pad)


def _dispatch_kernel(x_hbm, p1_hbm, p2_hbm, out_hbm,
                     idx1_v, idx2_v, rows_v, sem1, sem2):
    wid = lax.axis_index("s") * _NC + lax.axis_index("c")
    base = wid * _PER_W
    for c in range(_PER_W // _CH):
        b = base + c * _CH
        pltpu.sync_copy(p1_hbm.at[pl.ds(b, _CH)], idx1_v)
        pltpu.sync_copy(p2_hbm.at[pl.ds(b, _CH)], idx2_v)
        pltpu.sync_copy(x_hbm.at[pl.ds(b, _CH)], rows_v)
        cp1 = pltpu.make_async_copy(rows_v, out_hbm.at[idx1_v], sem1)
        cp2 = pltpu.make_async_copy(rows_v, out_hbm.at[idx2_v], sem2)
        cp1.start()
        cp2.start()
        cp1.wait()
        cp2.wait()


def _dispatch(hidden16, p1, p2):
    mesh = plsc.VectorSubcoreMesh(core_axis_name="c", subcore_axis_name="s")
    f = functools.partial(
        pl.kernel,
        out_type=jax.ShapeDtypeStruct((_S_PAD, _D), jnp.float32),
        mesh=mesh,
        scratch_types=[
            pltpu.VMEM((_CH,), jnp.int32),
            pltpu.VMEM((_CH,), jnp.int32),
            pltpu.VMEM((_CH, _D), jnp.float32),
            pltpu.SemaphoreType.DMA,
            pltpu.SemaphoreType.DMA,
        ],
    )(_dispatch_kernel)
    return f(hidden16, p1, p2)


_DK = _D // 2
_DQ = _D // 4


def _gemm1_kernel(be_ref, bv_ref, x_ref, wg0_ref, wg1_ref, wg2_ref, wg3_ref,
                  wu0_ref, wu1_ref, wu2_ref, wu3_ref, h_ref):
    b = pl.program_id(1)

    @pl.when(bv_ref[b] == 1)
    def _():
        wgs = (wg0_ref, wg1_ref, wg2_ref, wg3_ref)
        wus = (wu0_ref, wu1_ref, wu2_ref, wu3_ref)
        g = jnp.zeros((_BT, _BF), jnp.float32)
        u = jnp.zeros((_BT, _BF), jnp.float32)
        for q in range(4):
            xq = x_ref[:, q * _DQ:(q + 1) * _DQ]
            g = g + jnp.dot(xq, wgs[q][0], preferred_element_type=jnp.float32)
            u = u + jnp.dot(xq, wus[q][0], preferred_element_type=jnp.float32)
        h_ref[...] = (jax.nn.gelu(g) * u).astype(jnp.bfloat16)


def _gemm1(be, bv, x_sorted, w_gate, w_up):
    nf = _F // _BF
    wspecs = [pl.BlockSpec((1, _DQ, _BF),
                           (lambda q: lambda f, b, be, bv: (be[b], q, f))(q))
              for q in range(4)]
    grid_spec = pltpu.PrefetchScalarGridSpec(
        num_scalar_prefetch=2,
        grid=(nf, _NB),
        in_specs=[pl.BlockSpec((_BT, _D), lambda f, b, be, bv: (b, 0))]
        + wspecs + wspecs,
        out_specs=pl.BlockSpec((_BT, _BF), lambda f, b, be, bv: (b, f)),
    )
    return pl.pallas_call(
        _gemm1_kernel,
        grid_spec=grid_spec,
        out_shape=jax.ShapeDtypeStruct((_S_PAD, _F), jnp.bfloat16),
    )(be, bv, x_sorted, w_gate, w_gate, w_gate, w_gate, w_up, w_up, w_up, w_up)


def _gemm2_kernel(be_ref, bv_ref, h_ref, wd0_ref, wd1_ref, wd2_ref, wd3_ref,
                  y_ref):
    b = pl.program_id(0)

    @pl.when(bv_ref[b] == 1)
    def _():
        wds = (wd0_ref, wd1_ref, wd2_ref, wd3_ref)
        y = jnp.zeros((_BT, _D), jnp.float32)
        for q in range(4):
            hq = h_ref[:, q * _DQ:(q + 1) * _DQ]
            y = y + jnp.dot(hq, wds[q][0], preferred_element_type=jnp.float32)
        y_ref[...] = y


def _gemm2(be, bv, h, w_down):
    wspecs = [pl.BlockSpec((1, _DQ, _D),
                           (lambda q: lambda b, be, bv: (be[b], q, 0))(q))
              for q in range(4)]
    grid_spec = pltpu.PrefetchScalarGridSpec(
        num_scalar_prefetch=2,
        grid=(_NB,),
        in_specs=[pl.BlockSpec((_BT, _F), lambda b, be, bv: (b, 0))] + wspecs,
        out_specs=pl.BlockSpec((_BT, _D), lambda b, be, bv: (b, 0)),
    )
    return pl.pallas_call(
        _gemm2_kernel,
        grid_spec=grid_spec,
        out_shape=jax.ShapeDtypeStruct((_S_PAD, _D), jnp.float32),
    )(be, bv, h, w_down, w_down, w_down, w_down)


def _gather_kernel(y_hbm, p1_hbm, p2_hbm, y1_hbm, y2_hbm,
                   idx_v, rows_v, sem):
    wid = lax.axis_index("s") * _NC + lax.axis_index("c")
    base = wid * _PER_W
    for c in range(_PER_W // _CH):
        b = base + c * _CH
        pltpu.sync_copy(p1_hbm.at[pl.ds(b, _CH)], idx_v)
        pltpu.make_async_copy(y_hbm.at[idx_v], rows_v, sem).start()
        pltpu.make_async_copy(y_hbm.at[idx_v], rows_v, sem).wait()
        pltpu.sync_copy(rows_v, y1_hbm.at[pl.ds(b, _CH)])
        pltpu.sync_copy(p2_hbm.at[pl.ds(b, _CH)], idx_v)
        pltpu.make_async_copy(y_hbm.at[idx_v], rows_v, sem).start()
        pltpu.make_async_copy(y_hbm.at[idx_v], rows_v, sem).wait()
        pltpu.sync_copy(rows_v, y2_hbm.at[pl.ds(b, _CH)])


def _gather(y, p1, p2):
    mesh = plsc.VectorSubcoreMesh(core_axis_name="c", subcore_axis_name="s")
    f = functools.partial(
        pl.kernel,
        out_type=(
            jax.ShapeDtypeStruct((_T, _D), jnp.float32),
            jax.ShapeDtypeStruct((_T, _D), jnp.float32),
        ),
        mesh=mesh,
        scratch_types=[
            pltpu.VMEM((_CH,), jnp.int32),
            pltpu.VMEM((_CH, _D), jnp.float32),
            pltpu.SemaphoreType.DMA,
        ],
    )(_gather_kernel)
    return f(y, p1, p2)


def _combine_kernel(w1_ref, w2_ref, y1_ref, y2_ref, out_ref):
    out_ref[...] = w1_ref[...] * y1_ref[...] + w2_ref[...] * y2_ref[...]


def _combine(w1, w2, y1, y2):
    bt = 256
    grid_spec = pl.GridSpec(
        grid=(_T // bt,),
        in_specs=[
            pl.BlockSpec((bt, 1), lambda i: (i, 0)),
            pl.BlockSpec((bt, 1), lambda i: (i, 0)),
            pl.BlockSpec((bt, _D), lambda i: (i, 0)),
            pl.BlockSpec((bt, _D), lambda i: (i, 0)),
        ],
        out_specs=pl.BlockSpec((bt, _D), lambda i: (i, 0)),
    )
    return pl.pallas_call(
        _combine_kernel,
        grid_spec=grid_spec,
        out_shape=jax.ShapeDtypeStruct((_T, _D), jnp.float32),
    )(w1, w2, y1, y2)


def kernel(hidden_states, gate_w, w_gate, w_up, w_down):
    gw_pad = jnp.pad(gate_w, ((0, 0), (0, 128 - _E)))
    p1, p2, w1, w2, be, bv = _router(hidden_states, gw_pad)
    p1f = p1.reshape(_T)
    p2f = p2.reshape(_T)
    be = be.reshape(_NB_OUT)[:_NB]
    bv = bv.reshape(_NB_OUT)[:_NB]
    x_sorted = _dispatch(hidden_states, p1f, p2f)
    h = _gemm1(be, bv, x_sorted, w_gate, w_up)
    y = _gemm2(be, bv, h, w_down)
    y1, y2 = _gather(y, p1f, p2f)
    return _combine(w1, w2, y1, y2)


# speedup vs baseline: 1.0493x; 1.0102x over previous
import functools

import jax
import jax.numpy as jnp
from jax import lax
from jax.experimental import pallas as pl
from jax.experimental.pallas import tpu as pltpu
from jax.experimental.pallas import tpu_sc as plsc

_T = 2048
_D = 2048
_F = 2048
_E = 8
_SOFTCAP = 30.0

_BT = 128
_S_PAD = 2 * _T + _E * _BT
_NB = _S_PAD // _BT
_NB_OUT = 64
_BF = 1024

_NC = 2
_NS = 16
_NW = _NC * _NS
_PER_W = _T // _NW
_CH = 32


def _excl_cumsum_tokens(h):
    c = h
    s = 1
    while s < _T:
        z = jnp.zeros((s, 128), jnp.float32)
        c = c + jnp.concatenate([z, c[:-s]], axis=0)
        s *= 2
    return c - h


def _router_kernel(hid_ref, gw_ref, p1_ref, p2_ref, w1_ref, w2_ref,
                   be_ref, bv_ref):
    x = hid_ref[...]
    gw = gw_ref[...]
    logits = jnp.dot(x, gw, preferred_element_type=jnp.float32)
    lane = lax.broadcasted_iota(jnp.int32, (_T, 128), 1)
    neg = jnp.float32(-1e30)
    logits = jnp.where(lane < _E, _SOFTCAP * jnp.tanh(logits / _SOFTCAP), neg)
    v1 = jnp.max(logits, axis=-1, keepdims=True)
    i1 = jnp.min(jnp.where(logits == v1, lane, 127), axis=-1, keepdims=True)
    l2 = jnp.where(lane == i1, neg, logits)
    v2 = jnp.max(l2, axis=-1, keepdims=True)
    i2 = jnp.min(jnp.where(l2 == v2, lane, 127), axis=-1, keepdims=True)
    e2v = jnp.exp(v2 - v1)
    w1_ref[...] = 1.0 / (1.0 + e2v)
    w2_ref[...] = e2v / (1.0 + e2v)

    h1 = (lane == i1).astype(jnp.float32)
    h2 = (lane == i2).astype(jnp.float32)
    cnt1 = jnp.sum(h1, axis=0, keepdims=True)
    cnt2 = jnp.sum(h2, axis=0, keepdims=True)
    cnt = cnt1 + cnt2
    gpad = jnp.ceil(cnt / _BT) * _BT
    r_i = lax.broadcasted_iota(jnp.int32, (128, 128), 0)
    c_i = lax.broadcasted_iota(jnp.int32, (128, 128), 1)
    lt = (r_i < c_i).astype(jnp.float32)
    pad_off = jnp.dot(gpad, lt, preferred_element_type=jnp.float32)

    c1x = _excl_cumsum_tokens(h1)
    c2x = _excl_cumsum_tokens(h2)
    rank1 = jnp.sum(c1x * h1, axis=-1, keepdims=True)
    rank2 = jnp.sum(c2x * h2, axis=-1, keepdims=True)
    off1 = jnp.sum(pad_off * h1, axis=-1, keepdims=True)
    off2 = jnp.sum(pad_off * h2, axis=-1, keepdims=True)
    c1_at_e2 = jnp.sum(cnt1 * h2, axis=-1, keepdims=True)
    p1_ref[...] = (off1 + rank1).astype(jnp.int32)
    p2_ref[...] = (off2 + c1_at_e2 + rank2).astype(jnp.int32)

    b_start = (lax.broadcasted_iota(jnp.int32, (_NB_OUT, 128), 0)
               .astype(jnp.float32) * _BT)
    lane_e = lax.broadcasted_iota(jnp.int32, (_NB_OUT, 128), 1)
    sel = (b_start >= pad_off) & (b_start < pad_off + gpad) & (lane_e < _E)
    be = 7 - jnp.sum(jnp.where(sel, (7 - lane_e).astype(jnp.float32), 0.0),
                     axis=-1, keepdims=True)
    bv = jnp.sum(jnp.where(sel & (b_start < pad_off + cnt), 1.0, 0.0),
                 axis=-1, keepdims=True)
    be_ref[...] = be.astype(jnp.int32)
    bv_ref[...] = bv.astype(jnp.int32)


def _router(hidden, gate_w_pad):
    out_shapes = (
        jax.ShapeDtypeStruct((_T, 1), jnp.int32),
        jax.ShapeDtypeStruct((_T, 1), jnp.int32),
        jax.ShapeDtypeStruct((_T, 1), jnp.float32),
        jax.ShapeDtypeStruct((_T, 1), jnp.float32),
        jax.ShapeDtypeStruct((_NB_OUT, 1), jnp.int32),
        jax.ShapeDtypeStruct((_NB_OUT, 1), jnp.int32),
    )
    return pl.pallas_call(
        _router_kernel,
        out_shape=out_shapes,
    )(hidden, gate_w_pad)


def _dispatch_kernel(x_hbm, p1_hbm, p2_hbm, out_hbm,
                     idx1_v, idx2_v, rows_v, sem1, sem2):
    wid = lax.axis_index("s") * _NC + lax.axis_index("c")
    base = wid * _PER_W
    for c in range(_PER_W // _CH):
        b = base + c * _CH
        pltpu.sync_copy(p1_hbm.at[pl.ds(b, _CH)], idx1_v)
        pltpu.sync_copy(p2_hbm.at[pl.ds(b, _CH)], idx2_v)
        pltpu.sync_copy(x_hbm.at[pl.ds(b, _CH)], rows_v)
        cp1 = pltpu.make_async_copy(rows_v, out_hbm.at[idx1_v], sem1)
        cp2 = pltpu.make_async_copy(rows_v, out_hbm.at[idx2_v], sem2)
        cp1.start()
        cp2.start()
        cp1.wait()
        cp2.wait()


def _dispatch(hidden16, p1, p2):
    mesh = plsc.VectorSubcoreMesh(core_axis_name="c", subcore_axis_name="s")
    f = functools.partial(
        pl.kernel,
        out_type=jax.ShapeDtypeStruct((_S_PAD, _D), jnp.float32),
        mesh=mesh,
        scratch_types=[
            pltpu.VMEM((_CH,), jnp.int32),
            pltpu.VMEM((_CH,), jnp.int32),
            pltpu.VMEM((_CH, _D), jnp.float32),
            pltpu.SemaphoreType.DMA,
            pltpu.SemaphoreType.DMA,
        ],
    )(_dispatch_kernel)
    return f(hidden16, p1, p2)


_DK = _D // 2


def _gemm1_kernel(be_ref, bv_ref, x_ref, wga_ref, wgb_ref, wua_ref, wub_ref,
                  h_ref):
    b = pl.program_id(1)

    @pl.when(bv_ref[b] == 1)
    def _():
        xa = x_ref[:, :_DK]
        xb = x_ref[:, _DK:]
        g = (jnp.dot(xa, wga_ref[0], preferred_element_type=jnp.float32)
             + jnp.dot(xb, wgb_ref[0], preferred_element_type=jnp.float32))
        u = (jnp.dot(xa, wua_ref[0], preferred_element_type=jnp.float32)
             + jnp.dot(xb, wub_ref[0], preferred_element_type=jnp.float32))
        h_ref[...] = (jax.nn.gelu(g) * u).astype(jnp.bfloat16)


def _gemm1(be, bv, x_sorted, w_gate, w_up):
    nf = _F // _BF
    wspec_a = pl.BlockSpec((1, _DK, _BF), lambda f, b, be, bv: (be[b], 0, f))
    wspec_b = pl.BlockSpec((1, _DK, _BF), lambda f, b, be, bv: (be[b], 1, f))
    grid_spec = pltpu.PrefetchScalarGridSpec(
        num_scalar_prefetch=2,
        grid=(nf, _NB),
        in_specs=[
            pl.BlockSpec((_BT, _D), lambda f, b, be, bv: (b, 0)),
            wspec_a, wspec_b, wspec_a, wspec_b,
        ],
        out_specs=pl.BlockSpec((_BT, _BF), lambda f, b, be, bv: (b, f)),
    )
    return pl.pallas_call(
        _gemm1_kernel,
        grid_spec=grid_spec,
        out_shape=jax.ShapeDtypeStruct((_S_PAD, _F), jnp.bfloat16),
    )(be, bv, x_sorted, w_gate, w_gate, w_up, w_up)


def _gemm2_kernel(be_ref, bv_ref, h_ref, wda_ref, wdb_ref, y_ref):
    b = pl.program_id(0)

    @pl.when(bv_ref[b] == 1)
    def _():
        y_ref[...] = (
            jnp.dot(h_ref[:, :_DK], wda_ref[0],
                    preferred_element_type=jnp.float32)
            + jnp.dot(h_ref[:, _DK:], wdb_ref[0],
                      preferred_element_type=jnp.float32)
        )


def _gemm2(be, bv, h, w_down):
    wspec_a = pl.BlockSpec((1, _DK, _D), lambda b, be, bv: (be[b], 0, 0))
    wspec_b = pl.BlockSpec((1, _DK, _D), lambda b, be, bv: (be[b], 1, 0))
    grid_spec = pltpu.PrefetchScalarGridSpec(
        num_scalar_prefetch=2,
        grid=(_NB,),
        in_specs=[
            pl.BlockSpec((_BT, _F), lambda b, be, bv: (b, 0)),
            wspec_a, wspec_b,
        ],
        out_specs=pl.BlockSpec((_BT, _D), lambda b, be, bv: (b, 0)),
    )
    return pl.pallas_call(
        _gemm2_kernel,
        grid_spec=grid_spec,
        out_shape=jax.ShapeDtypeStruct((_S_PAD, _D), jnp.float32),
    )(be, bv, h, w_down, w_down)


def _gather_kernel(y_hbm, p1_hbm, p2_hbm, y1_hbm, y2_hbm,
                   idx_v, rows_v, sem):
    wid = lax.axis_index("s") * _NC + lax.axis_index("c")
    base = wid * _PER_W
    for c in range(_PER_W // _CH):
        b = base + c * _CH
        pltpu.sync_copy(p1_hbm.at[pl.ds(b, _CH)], idx_v)
        pltpu.make_async_copy(y_hbm.at[idx_v], rows_v, sem).start()
        pltpu.make_async_copy(y_hbm.at[idx_v], rows_v, sem).wait()
        pltpu.sync_copy(rows_v, y1_hbm.at[pl.ds(b, _CH)])
        pltpu.sync_copy(p2_hbm.at[pl.ds(b, _CH)], idx_v)
        pltpu.make_async_copy(y_hbm.at[idx_v], rows_v, sem).start()
        pltpu.make_async_copy(y_hbm.at[idx_v], rows_v, sem).wait()
        pltpu.sync_copy(rows_v, y2_hbm.at[pl.ds(b, _CH)])


def _gather(y, p1, p2):
    mesh = plsc.VectorSubcoreMesh(core_axis_name="c", subcore_axis_name="s")
    f = functools.partial(
        pl.kernel,
        out_type=(
            jax.ShapeDtypeStruct((_T, _D), jnp.float32),
            jax.ShapeDtypeStruct((_T, _D), jnp.float32),
        ),
        mesh=mesh,
        scratch_types=[
            pltpu.VMEM((_CH,), jnp.int32),
            pltpu.VMEM((_CH, _D), jnp.float32),
            pltpu.SemaphoreType.DMA,
        ],
    )(_gather_kernel)
    return f(y, p1, p2)


def _combine_kernel(w1_ref, w2_ref, y1_ref, y2_ref, out_ref):
    out_ref[...] = w1_ref[...] * y1_ref[...] + w2_ref[...] * y2_ref[...]


def _combine(w1, w2, y1, y2):
    bt = 256
    grid_spec = pl.GridSpec(
        grid=(_T // bt,),
        in_specs=[
            pl.BlockSpec((bt, 1), lambda i: (i, 0)),
            pl.BlockSpec((bt, 1), lambda i: (i, 0)),
            pl.BlockSpec((bt, _D), lambda i: (i, 0)),
            pl.BlockSpec((bt, _D), lambda i: (i, 0)),
        ],
        out_specs=pl.BlockSpec((bt, _D), lambda i: (i, 0)),
    )
    return pl.pallas_call(
        _combine_kernel,
        grid_spec=grid_spec,
        out_shape=jax.ShapeDtypeStruct((_T, _D), jnp.float32),
    )(w1, w2, y1, y2)


def kernel(hidden_states, gate_w, w_gate, w_up, w_down):
    gw_pad = jnp.pad(gate_w, ((0, 0), (0, 128 - _E)))
    p1, p2, w1, w2, be, bv = _router(hidden_states, gw_pad)
    p1f = p1.reshape(_T)
    p2f = p2.reshape(_T)
    be = be.reshape(_NB_OUT)[:_NB]
    bv = bv.reshape(_NB_OUT)[:_NB]
    x_sorted = _dispatch(hidden_states, p1f, p2f)
    h = _gemm1(be, bv, x_sorted, w_gate, w_up)
    y = _gemm2(be, bv, h, w_down)
    y1, y2 = _gather(y, p1f, p2f)
    return _combine(w1, w2, y1, y2)
